# Initial kernel scaffold; baseline (speedup 1.0000x reference)
import jax
import jax.numpy as jnp
from jax.experimental import pallas as pl


def kernel(x):
    raise NotImplementedError("write your pallas kernel here")


# speedup vs baseline: 5.6816x; 5.6816x over previous
import functools

import jax
import jax.numpy as jnp
from jax import lax
from jax.experimental import pallas as pl
from jax.experimental.pallas import tpu as pltpu
from jax.experimental.pallas import tpu_sc as plsc

ROWS = 4096
COLS = 32768
KTOP = 512
NC = 2
NS = 16
L = 16
NW = NC * NS
NVEC = COLS // L


def _sc_threshold_body(x_hbm, thr_hbm, row_buf, hist, cnt_smem, thr_vec):
  cid = lax.axis_index("c")
  sid = lax.axis_index("s")
  wid = sid * NC + cid

  zeros16 = jnp.zeros((L,), jnp.int32)
  ones16 = jnp.ones((L,), jnp.int32)
  lanes = lax.iota(jnp.int32, L)

  def zero_hist(b, carry):
    hist[b] = zeros16
    return carry

  lax.fori_loop(0, 256, zero_hist, 0)

  def key_at(i):
    v = row_buf[pl.ds(i * L, L)]
    b = plsc.bitcast(v, jnp.int32)
    return jnp.where(b >= 0, b, b ^ jnp.int32(0x7FFFFFFF))

  def do_row(j, carry):
    row = wid + j * NW
    pltpu.sync_copy(x_hbm.at[row], row_buf)

    prefix = jnp.int32(0)
    rank = jnp.int32(KTOP)
    for sh in (24, 16, 8, 0):

      def hist_body(i, c, sh=sh, prefix=prefix):
        s = key_at(i)
        if sh == 24:
          bin_ = (s >> 24) + 128
          plsc.addupdate_scatter(hist, [bin_, lanes], ones16)
        else:
          bin_ = (s >> sh) & 0xFF
          m = (s >> (sh + 8)) == prefix
          plsc.addupdate_scatter(hist, [bin_, lanes], ones16, mask=m)
        return c

      lax.fori_loop(0, NVEC, hist_body, 0)

      def sums_body(b, c):
        cnt_smem[b] = jnp.sum(hist[b])
        hist[b] = zeros16
        return c

      lax.fori_loop(0, 256, sums_body, 0)

      def cond(c):
        b, acc = c
        return acc + cnt_smem[b] < rank

      def body(c):
        b, acc = c
        return b - 1, acc + cnt_smem[b]

      bstar, acc_above = lax.while_loop(cond, body,
                                        (jnp.int32(255), jnp.int32(0)))
      rank = rank - acc_above
      if sh == 24:
        prefix = bstar - 128
      else:
        prefix = prefix * 256 + bstar

    t = jnp.maximum(prefix, 1)
    thr_vec[...] = jnp.full((L,), 0, jnp.int32) + t
    pltpu.sync_copy(thr_vec, thr_hbm.at[row])
    return carry

  lax.fori_loop(0, ROWS // NW, do_row, 0)


_sc_threshold = functools.partial(
    pl.kernel,
    out_type=jax.ShapeDtypeStruct((ROWS, L), jnp.int32),
    mesh=plsc.VectorSubcoreMesh(
        core_axis_name="c", subcore_axis_name="s",
        num_cores=NC, num_subcores=NS),
    scratch_types=[
        pltpu.VMEM((COLS,), jnp.float32),
        pltpu.VMEM((256, L), jnp.int32),
        pltpu.SMEM((256,), jnp.int32),
        pltpu.VMEM((L,), jnp.int32),
    ],
    compiler_params=pltpu.CompilerParams(needs_layout_passes=False),
)(_sc_threshold_body)


_BR = 8


def _mask_body(x_ref, thr_ref, o_ref):
  x = x_ref[...]
  b = lax.bitcast_convert_type(x, jnp.int32)
  s = jnp.where(b >= 0, b, b ^ jnp.int32(0x7FFFFFFF))
  t = thr_ref[:, 0:1]
  o_ref[...] = jnp.where(s >= t, x, jnp.float32(0.0))


def kernel(x):
  thr = _sc_threshold(x)
  out = pl.pallas_call(
      _mask_body,
      grid=(ROWS // _BR,),
      in_specs=[
          pl.BlockSpec((_BR, COLS), lambda i: (i, 0)),
          pl.BlockSpec((_BR, L), lambda i: (i, 0)),
      ],
      out_specs=pl.BlockSpec((_BR, COLS), lambda i: (i, 0)),
      out_shape=jax.ShapeDtypeStruct((ROWS, COLS), jnp.float32),
  )(x, thr)
  return out


# speedup vs baseline: 7.3800x; 1.2989x over previous
import functools

import jax
import jax.numpy as jnp
from jax import lax
from jax.experimental import pallas as pl
from jax.experimental.pallas import tpu as pltpu
from jax.experimental.pallas import tpu_sc as plsc

ROWS = 4096
COLS = 32768
KTOP = 512
NC = 2
NS = 16
L = 16
NW = NC * NS
NVEC = COLS // L


def _sc_threshold_body(x_hbm, thr_hbm, row_buf, cand, hist, cnt, thr_vec):
  cid = lax.axis_index("c")
  sid = lax.axis_index("s")
  wid = sid * NC + cid

  zeros16 = jnp.zeros((L,), jnp.int32)
  ones16 = jnp.ones((L,), jnp.int32)
  lanes = lax.iota(jnp.int32, L)

  def zero_hist(r, c):
    for g in range(16):
      hist[r, pl.ds(g * L, L)] = zeros16
    return c

  lax.fori_loop(0, 16, zero_hist, 0)

  def col_sums(rank):
    def cs(g, c):
      acc = zeros16
      for r in range(16):
        acc = acc + hist[r, pl.ds(g * L, L)]
        hist[r, pl.ds(g * L, L)] = zeros16
      cnt[pl.ds(g * L, L)] = acc
      return c

    lax.fori_loop(0, 16, cs, 0)

    def cond(cs_):
      b, acc = cs_
      return acc + cnt[pl.ds(b, L)][0] < rank

    def body(cs_):
      b, acc = cs_
      return b - 1, acc + cnt[pl.ds(b, L)][0]

    bstar, acc_above = lax.while_loop(cond, body,
                                      (jnp.int32(255), jnp.int32(0)))
    return bstar, rank - acc_above

  def do_row(j, carry):
    row = wid + j * NW
    pltpu.sync_copy(x_hbm.at[row], row_buf)

    def h1(i, c):
      v = row_buf[pl.ds(i * L, L)]
      bu = plsc.bitcast(v, jnp.int32)
      raw = lax.shift_right_logical(bu, 24)
      plsc.addupdate_scatter(hist, [lanes, raw], ones16)
      return c

    lax.fori_loop(0, NVEC, h1, 0, unroll=8)

    def cs(g, c):
      acc = zeros16
      for r in range(16):
        acc = acc + hist[r, pl.ds(g * L, L)]
        hist[r, pl.ds(g * L, L)] = zeros16
      cnt[pl.ds(g * L, L)] = acc
      return c

    lax.fori_loop(0, 16, cs, 0)

    def cond1(st):
      k, acc = st
      b = jnp.where(k < 128, 127 - k, k)
      return acc + cnt[pl.ds(b, L)][0] < KTOP

    def body1(st):
      k, acc = st
      b = jnp.where(k < 128, 127 - k, k)
      return k + 1, acc + cnt[pl.ds(b, L)][0]

    kstar, acc_above = lax.while_loop(cond1, body1,
                                      (jnp.int32(0), jnp.int32(0)))
    b1raw = jnp.where(kstar < 128, 127 - kstar, kstar)
    rank = jnp.int32(KTOP) - acc_above
    prefix = jnp.where(b1raw < 128, b1raw, 127 - b1raw)

    def cp(i, off):
      v = row_buf[pl.ds(i * L, L)]
      bu = plsc.bitcast(v, jnp.int32)
      raw = lax.shift_right_logical(bu, 24)
      m = raw == b1raw
      s = jnp.where(bu >= 0, bu, bu ^ jnp.int32(0x7FFFFFFF))
      plsc.store_compressed(cand.at[pl.ds(off, L)], s, mask=m)
      return off + jnp.sum(m.astype(jnp.int32))

    ncand = lax.fori_loop(0, NVEC, cp, jnp.int32(0), unroll=4)
    nv = (ncand + (L - 1)) // L

    def round_body(ri, st):
      prefix, rank = st
      sh = 16 - 8 * ri

      def hb(i, c):
        s = cand[pl.ds(i * L, L)]
        valid = (i * L + lanes) < ncand
        m = valid & (lax.shift_right_arithmetic(s, sh + 8) == prefix)
        bin_ = lax.shift_right_logical(s, sh) & jnp.int32(0xFF)
        plsc.addupdate_scatter(hist, [lanes, bin_], ones16, mask=m)
        return c

      lax.fori_loop(0, nv, hb, 0)
      bstar, rank = col_sums(rank)
      return prefix * 256 + bstar, rank

    prefix, rank = lax.fori_loop(0, 3, round_body, (prefix, rank))

    t = jnp.maximum(prefix, 1)
    thr_vec[...] = jnp.full((L,), 0, jnp.int32) + t
    pltpu.sync_copy(thr_vec, thr_hbm.at[row])
    return carry

  lax.fori_loop(0, ROWS // NW, do_row, 0)


_sc_threshold = functools.partial(
    pl.kernel,
    out_type=jax.ShapeDtypeStruct((ROWS, L), jnp.int32),
    mesh=plsc.VectorSubcoreMesh(
        core_axis_name="c", subcore_axis_name="s",
        num_cores=NC, num_subcores=NS),
    scratch_types=[
        pltpu.VMEM((COLS,), jnp.float32),
        pltpu.VMEM((COLS,), jnp.int32),
        pltpu.VMEM((16, 256), jnp.int32),
        pltpu.VMEM((256 + L,), jnp.int32),
        pltpu.VMEM((L,), jnp.int32),
    ],
    compiler_params=pltpu.CompilerParams(needs_layout_passes=False),
)(_sc_threshold_body)


_BR = 8


def _mask_body(x_ref, thr_ref, o_ref):
  x = x_ref[...]
  b = lax.bitcast_convert_type(x, jnp.int32)
  s = jnp.where(b >= 0, b, b ^ jnp.int32(0x7FFFFFFF))
  t = thr_ref[:, 0:1]
  o_ref[...] = jnp.where(s >= t, x, jnp.float32(0.0))


def kernel(x):
  thr = _sc_threshold(x)
  out = pl.pallas_call(
      _mask_body,
      grid=(ROWS // _BR,),
      in_specs=[
          pl.BlockSpec((_BR, COLS), lambda i: (i, 0)),
          pl.BlockSpec((_BR, L), lambda i: (i, 0)),
      ],
      out_specs=pl.BlockSpec((_BR, COLS), lambda i: (i, 0)),
      out_shape=jax.ShapeDtypeStruct((ROWS, COLS), jnp.float32),
  )(x, thr)
  return out


# speedup vs baseline: 19.0336x; 2.5791x over previous
import functools

import jax
import jax.numpy as jnp
from jax import lax
from jax.experimental import pallas as pl
from jax.experimental.pallas import tpu as pltpu
from jax.experimental.pallas import tpu_sc as plsc

ROWS = 4096
COLS = 32768
KTOP = 512
NC = 2
NS = 16
L = 16
NW = NC * NS
NVEC = COLS // L


def _sc_threshold_body(x_hbm, thr_hbm, row_a, row_b, cand, hist, cnt,
                       thr_vec, sem_a, sem_b):
  cid = lax.axis_index("c")
  sid = lax.axis_index("s")
  wid = sid * NC + cid

  zeros16 = jnp.zeros((L,), jnp.int32)
  ones16 = jnp.ones((L,), jnp.int32)
  lanes = lax.iota(jnp.int32, L)

  def zero_hist(r, c):
    for g in range(16):
      hist[r, pl.ds(g * L, L)] = zeros16
    return c

  lax.fori_loop(0, 16, zero_hist, 0)

  def process_row(row_buf, row):
    @plsc.parallel_loop(0, NVEC, unroll=8)
    def _(i):
      v = row_buf[pl.ds(i * L, L)]
      bu = plsc.bitcast(v, jnp.int32)
      raw = lax.shift_right_logical(bu, 24)
      plsc.addupdate_scatter(hist, [lanes, raw], ones16)

    def cs(g, c):
      acc = zeros16
      for r in range(16):
        acc = acc + hist[r, pl.ds(g * L, L)]
        hist[r, pl.ds(g * L, L)] = zeros16
      cnt[pl.ds(g * L, L)] = acc
      return c

    lax.fori_loop(0, 16, cs, 0)

    def cond1(st):
      k, acc = st
      b = jnp.where(k < 128, 127 - k, k)
      return acc + cnt[pl.ds(b, L)][0] < KTOP

    def body1(st):
      k, acc = st
      b = jnp.where(k < 128, 127 - k, k)
      return k + 1, acc + cnt[pl.ds(b, L)][0]

    kstar, acc_above = lax.while_loop(cond1, body1,
                                      (jnp.int32(0), jnp.int32(0)))
    b1raw = jnp.where(kstar < 128, 127 - kstar, kstar)
    rank = jnp.int32(KTOP) - acc_above
    prefix = jnp.where(b1raw < 128, b1raw, 127 - b1raw)

    @plsc.parallel_loop(0, NVEC, unroll=4, carry=jnp.int32(0))
    def ncand(i, off):
      v = row_buf[pl.ds(i * L, L)]
      bu = plsc.bitcast(v, jnp.int32)
      raw = lax.shift_right_logical(bu, 24)
      m = raw == b1raw
      s = jnp.where(bu >= 0, bu, bu ^ jnp.int32(0x7FFFFFFF))
      plsc.store_compressed(cand.at[pl.ds(off, L)], s, mask=m)
      return off + plsc.all_reduce_population_count(m)[0]

    nv = (ncand + (L - 1)) // L

    def round_body(ri, st):
      prefix, rank = st
      sh = 20 - 4 * ri

      def hb(i, c):
        s = cand[pl.ds(i * L, L)]
        valid = (i * L + lanes) < ncand
        m = valid & (lax.shift_right_arithmetic(s, sh + 4) == prefix)
        bin_ = lax.shift_right_logical(s, sh) & jnp.int32(0xF)
        plsc.addupdate_scatter(hist, [lanes, bin_], ones16, mask=m)
        return c

      lax.fori_loop(0, nv, hb, 0)

      acc = zeros16
      for r in range(16):
        acc = acc + hist[r, pl.ds(0, L)]
        hist[r, pl.ds(0, L)] = zeros16
      cnt[pl.ds(0, L)] = acc

      def cond(cs_):
        b, a = cs_
        return a + cnt[pl.ds(b, L)][0] < rank

      def body(cs_):
        b, a = cs_
        return b - 1, a + cnt[pl.ds(b, L)][0]

      bstar, acc_ab = lax.while_loop(cond, body,
                                     (jnp.int32(15), jnp.int32(0)))
      return prefix * 16 + bstar, rank - acc_ab

    prefix, rank = lax.fori_loop(0, 6, round_body, (prefix, rank))

    t = jnp.maximum(prefix, 1)
    thr_vec[...] = jnp.full((L,), 0, jnp.int32) + t
    pltpu.sync_copy(thr_vec, thr_hbm.at[row])

  nrows = ROWS // NW
  pltpu.async_copy(x_hbm.at[wid], row_a, sem_a)

  def do_pair(jj, carry):
    r0 = wid + (2 * jj) * NW
    r1 = r0 + NW
    pltpu.make_async_copy(x_hbm.at[r0], row_a, sem_a).wait()
    pltpu.async_copy(x_hbm.at[r1], row_b, sem_b)
    process_row(row_a, r0)
    pltpu.make_async_copy(x_hbm.at[r1], row_b, sem_b).wait()

    @pl.when(jj < nrows // 2 - 1)
    def _():
      pltpu.async_copy(x_hbm.at[r0 + 2 * NW], row_a, sem_a)

    process_row(row_b, r1)
    return carry

  lax.fori_loop(0, nrows // 2, do_pair, 0)


_sc_threshold = functools.partial(
    pl.kernel,
    out_type=jax.ShapeDtypeStruct((ROWS, L), jnp.int32),
    mesh=plsc.VectorSubcoreMesh(
        core_axis_name="c", subcore_axis_name="s",
        num_cores=NC, num_subcores=NS),
    scratch_types=[
        pltpu.VMEM((COLS,), jnp.float32),
        pltpu.VMEM((COLS,), jnp.float32),
        pltpu.VMEM((COLS,), jnp.int32),
        pltpu.VMEM((16, 256), jnp.int32),
        pltpu.VMEM((256 + L,), jnp.int32),
        pltpu.VMEM((L,), jnp.int32),
        pltpu.SemaphoreType.DMA,
        pltpu.SemaphoreType.DMA,
    ],
    compiler_params=pltpu.CompilerParams(needs_layout_passes=False),
)(_sc_threshold_body)


_BR = 8


def _mask_body(x_ref, thr_ref, o_ref):
  x = x_ref[...]
  b = lax.bitcast_convert_type(x, jnp.int32)
  s = jnp.where(b >= 0, b, b ^ jnp.int32(0x7FFFFFFF))
  t = thr_ref[:, 0:1]
  o_ref[...] = jnp.where(s >= t, x, jnp.float32(0.0))


def kernel(x):
  thr = _sc_threshold(x)
  out = pl.pallas_call(
      _mask_body,
      grid=(ROWS // _BR,),
      in_specs=[
          pl.BlockSpec((_BR, COLS), lambda i: (i, 0)),
          pl.BlockSpec((_BR, L), lambda i: (i, 0)),
      ],
      out_specs=pl.BlockSpec((_BR, COLS), lambda i: (i, 0)),
      out_shape=jax.ShapeDtypeStruct((ROWS, COLS), jnp.float32),
  )(x, thr)
  return out


# speedup vs baseline: 37.8458x; 1.9884x over previous
import functools

import jax
import jax.numpy as jnp
from jax import lax
from jax.experimental import pallas as pl
from jax.experimental.pallas import tpu as pltpu
from jax.experimental.pallas import tpu_sc as plsc

ROWS = 4096
COLS = 32768
KTOP = 512
NC = 2
NS = 16
L = 16
NW = NC * NS
NVEC = COLS // L
RPW = ROWS // NW


def _sc_threshold_body(x_hbm, thr_hbm, row_a, row_b, cand, hist, thr_loc,
                       cnt, sem_a, sem_b):
  cid = lax.axis_index("c")
  sid = lax.axis_index("s")
  wid = sid * NC + cid

  zeros16 = jnp.zeros((L,), jnp.int32)
  ones16 = jnp.ones((L,), jnp.int32)
  lanes = lax.iota(jnp.int32, L)

  @plsc.parallel_loop(0, 256, unroll=4)
  def _(b):
    hist[pl.ds(b * L, L)] = zeros16

  def process_row(row_buf, j):
    @plsc.parallel_loop(0, NVEC, unroll=8)
    def _(i):
      v = row_buf[pl.ds(i * L, L)]
      bu = plsc.bitcast(v, jnp.int32)
      idx = (lax.shift_right_logical(bu, 20) & jnp.int32(0xFF0)) | lanes
      plsc.addupdate_scatter(hist, [idx], ones16)

    @plsc.parallel_loop(0, 256, unroll=4)
    def _(b):
      cnt[b] = jnp.sum(hist[pl.ds(b * L, L)])
      hist[pl.ds(b * L, L)] = zeros16

    def cond1(st):
      k, acc = st
      b = jnp.where(k < 128, 127 - k, k)
      return acc + cnt[b] < KTOP

    def body1(st):
      k, acc = st
      b = jnp.where(k < 128, 127 - k, k)
      return k + 1, acc + cnt[b]

    kstar, acc_above = lax.while_loop(cond1, body1,
                                      (jnp.int32(0), jnp.int32(0)))
    b1raw = jnp.where(kstar < 128, 127 - kstar, kstar)
    rank = jnp.int32(KTOP) - acc_above
    prefix = jnp.where(b1raw < 128, b1raw, 127 - b1raw)

    @plsc.parallel_loop(0, NVEC, unroll=4, carry=jnp.int32(0))
    def ncand(i, off):
      v = row_buf[pl.ds(i * L, L)]
      bu = plsc.bitcast(v, jnp.int32)
      raw = lax.shift_right_logical(bu, 24)
      m = raw == b1raw
      s = jnp.where(bu >= 0, bu, bu ^ jnp.int32(0x7FFFFFFF))
      plsc.store_compressed(cand.at[pl.ds(off, L)], s, mask=m)
      return off + plsc.all_reduce_population_count(m)[0]

    nv = (ncand + (L - 1)) // L

    def round_body(ri, st):
      prefix, rank = st
      sh = 20 - 4 * ri

      @plsc.parallel_loop(0, nv, unroll=2)
      def _(i):
        s = cand[pl.ds(i * L, L)]
        valid = (i * L + lanes) < ncand
        m = valid & (lax.shift_right_arithmetic(s, sh + 4) == prefix)
        idx = ((lax.shift_right_logical(s, sh) & jnp.int32(0xF)) * L) | lanes
        plsc.addupdate_scatter(hist, [idx], ones16, mask=m)

      for b in range(16):
        cnt[b] = jnp.sum(hist[pl.ds(b * L, L)])
        hist[pl.ds(b * L, L)] = zeros16

      def cond(cs_):
        b, a = cs_
        return a + cnt[b] < rank

      def body(cs_):
        b, a = cs_
        return b - 1, a + cnt[b]

      bstar, acc_ab = lax.while_loop(cond, body,
                                     (jnp.int32(15), jnp.int32(0)))
      return prefix * 16 + bstar, rank - acc_ab

    prefix, rank = lax.fori_loop(0, 6, round_body, (prefix, rank))

    t = jnp.maximum(prefix, 1)
    thr_loc[pl.ds(j * L, L)] = jnp.full((L,), 0, jnp.int32) + t

  pltpu.async_copy(x_hbm.at[wid], row_a, sem_a)

  def do_pair(jj, carry):
    j0 = 2 * jj
    r0 = wid + j0 * NW
    r1 = r0 + NW
    pltpu.make_async_copy(x_hbm.at[r0], row_a, sem_a).wait()
    pltpu.async_copy(x_hbm.at[r1], row_b, sem_b)
    process_row(row_a, j0)
    pltpu.make_async_copy(x_hbm.at[r1], row_b, sem_b).wait()

    @pl.when(jj < RPW // 2 - 1)
    def _():
      pltpu.async_copy(x_hbm.at[r0 + 2 * NW], row_a, sem_a)

    process_row(row_b, j0 + 1)
    return carry

  lax.fori_loop(0, RPW // 2, do_pair, 0)
  pltpu.sync_copy(thr_loc, thr_hbm.at[wid])


_sc_threshold = functools.partial(
    pl.kernel,
    out_type=jax.ShapeDtypeStruct((NW, RPW * L), jnp.int32),
    mesh=plsc.VectorSubcoreMesh(
        core_axis_name="c", subcore_axis_name="s",
        num_cores=NC, num_subcores=NS),
    scratch_types=[
        pltpu.VMEM((COLS,), jnp.float32),
        pltpu.VMEM((COLS,), jnp.float32),
        pltpu.VMEM((COLS,), jnp.int32),
        pltpu.VMEM((256 * L,), jnp.int32),
        pltpu.VMEM((RPW * L,), jnp.int32),
        pltpu.SMEM((256,), jnp.int32),
        pltpu.SemaphoreType.DMA,
        pltpu.SemaphoreType.DMA,
    ],
    compiler_params=pltpu.CompilerParams(needs_layout_passes=False),
)(_sc_threshold_body)


_BR = 8


def _mask_body(x_ref, thr_ref, o_ref):
  x = x_ref[...]
  b = lax.bitcast_convert_type(x, jnp.int32)
  s = jnp.where(b >= 0, b, b ^ jnp.int32(0x7FFFFFFF))
  t = thr_ref[:, 0:1]
  o_ref[...] = jnp.where(s >= t, x, jnp.float32(0.0))


def kernel(x):
  thr = _sc_threshold(x)
  thr_rows = thr.reshape(NW, RPW, L).transpose(1, 0, 2).reshape(ROWS, L)
  out = pl.pallas_call(
      _mask_body,
      grid=(ROWS // _BR,),
      in_specs=[
          pl.BlockSpec((_BR, COLS), lambda i: (i, 0)),
          pl.BlockSpec((_BR, L), lambda i: (i, 0)),
      ],
      out_specs=pl.BlockSpec((_BR, COLS), lambda i: (i, 0)),
      out_shape=jax.ShapeDtypeStruct((ROWS, COLS), jnp.float32),
  )(x, thr_rows)
  return out


# speedup vs baseline: 46.3304x; 1.2242x over previous
import functools

import jax
import jax.numpy as jnp
from jax import lax
from jax.experimental import pallas as pl
from jax.experimental.pallas import tpu as pltpu
from jax.experimental.pallas import tpu_sc as plsc

ROWS = 4096
COLS = 32768
KTOP = 512
NC = 2
NS = 16
L = 16
NW = NC * NS
NVEC = COLS // L
RPW = ROWS // NW


def _topk_body(x_hbm, out_hbm, row_a, row_b, cand, hist, cnt,
               sem_ia, sem_ib, sem_oa, sem_ob):
  cid = lax.axis_index("c")
  sid = lax.axis_index("s")
  wid = sid * NC + cid

  zeros16 = jnp.zeros((L,), jnp.int32)
  ones16 = jnp.ones((L,), jnp.int32)
  lanes = lax.iota(jnp.int32, L)

  @plsc.parallel_loop(0, 256, unroll=4)
  def _(b):
    hist[pl.ds(b * L, L)] = zeros16

  def threshold(row_buf):
    @plsc.parallel_loop(0, NVEC, unroll=8)
    def _(i):
      v = row_buf[pl.ds(i * L, L)]
      bu = plsc.bitcast(v, jnp.int32)
      idx = (lax.shift_right_logical(bu, 20) & jnp.int32(0xFF0)) | lanes
      plsc.addupdate_scatter(hist, [idx], ones16)

    @plsc.parallel_loop(0, 256, unroll=4)
    def _(b):
      cnt[b] = jnp.sum(hist[pl.ds(b * L, L)])
      hist[pl.ds(b * L, L)] = zeros16

    def cond1(st):
      k, acc = st
      b = jnp.where(k < 128, 127 - k, k)
      return acc + cnt[b] < KTOP

    def body1(st):
      k, acc = st
      b = jnp.where(k < 128, 127 - k, k)
      return k + 1, acc + cnt[b]

    kstar, acc_above = lax.while_loop(cond1, body1,
                                      (jnp.int32(0), jnp.int32(0)))
    b1raw = jnp.where(kstar < 128, 127 - kstar, kstar)
    rank = jnp.int32(KTOP) - acc_above
    prefix = jnp.where(b1raw < 128, b1raw, 127 - b1raw)

    @plsc.parallel_loop(0, NVEC, unroll=4, carry=jnp.int32(0))
    def ncand(i, off):
      v = row_buf[pl.ds(i * L, L)]
      bu = plsc.bitcast(v, jnp.int32)
      raw = lax.shift_right_logical(bu, 24)
      m = raw == b1raw
      s = jnp.where(bu >= 0, bu, bu ^ jnp.int32(0x7FFFFFFF))
      plsc.store_compressed(cand.at[pl.ds(off, L)], s, mask=m)
      return off + plsc.all_reduce_population_count(m)[0]

    nv = (ncand + (L - 1)) // L

    def round_body(ri, st):
      prefix, rank = st
      sh = 20 - 4 * ri

      @plsc.parallel_loop(0, nv, unroll=2)
      def _(i):
        s = cand[pl.ds(i * L, L)]
        valid = (i * L + lanes) < ncand
        m = valid & (lax.shift_right_arithmetic(s, sh + 4) == prefix)
        idx = ((lax.shift_right_logical(s, sh) & jnp.int32(0xF)) * L) | lanes
        plsc.addupdate_scatter(hist, [idx], ones16, mask=m)

      for b in range(16):
        cnt[b] = jnp.sum(hist[pl.ds(b * L, L)])
        hist[pl.ds(b * L, L)] = zeros16

      def cond(cs_):
        b, a = cs_
        return a + cnt[b] < rank

      def body(cs_):
        b, a = cs_
        return b - 1, a + cnt[b]

      bstar, acc_ab = lax.while_loop(cond, body,
                                     (jnp.int32(15), jnp.int32(0)))
      return prefix * 16 + bstar, rank - acc_ab

    prefix, rank = lax.fori_loop(0, 6, round_body, (prefix, rank))
    return jnp.maximum(prefix, 1)

  def mask_row(row_buf, t):
    @plsc.parallel_loop(0, NVEC, unroll=8)
    def _(i):
      v = row_buf[pl.ds(i * L, L)]
      bu = plsc.bitcast(v, jnp.int32)
      row_buf[pl.ds(i * L, L)] = jnp.where(bu >= t, v, jnp.float32(0.0))

  pltpu.async_copy(x_hbm.at[wid], row_a, sem_ia)

  def do_pair(jj, carry):
    r0 = wid + (2 * jj) * NW
    r1 = r0 + NW
    pltpu.make_async_copy(x_hbm.at[r0], row_a, sem_ia).wait()

    @pl.when(jj > 0)
    def _():
      pltpu.make_async_copy(row_b, out_hbm.at[r1 - 2 * NW], sem_ob).wait()

    pltpu.async_copy(x_hbm.at[r1], row_b, sem_ib)

    t0 = threshold(row_a)
    mask_row(row_a, t0)
    pltpu.async_copy(row_a, out_hbm.at[r0], sem_oa)

    pltpu.make_async_copy(x_hbm.at[r1], row_b, sem_ib).wait()
    t1 = threshold(row_b)

    pltpu.make_async_copy(row_a, out_hbm.at[r0], sem_oa).wait()

    @pl.when(jj < RPW // 2 - 1)
    def _():
      pltpu.async_copy(x_hbm.at[r0 + 2 * NW], row_a, sem_ia)

    mask_row(row_b, t1)
    pltpu.async_copy(row_b, out_hbm.at[r1], sem_ob)
    return carry

  lax.fori_loop(0, RPW // 2, do_pair, 0)
  last_r1 = wid + (RPW - 1) * NW
  pltpu.make_async_copy(row_b, out_hbm.at[last_r1], sem_ob).wait()


kernel = functools.partial(
    pl.kernel,
    out_type=jax.ShapeDtypeStruct((ROWS, COLS), jnp.float32),
    mesh=plsc.VectorSubcoreMesh(
        core_axis_name="c", subcore_axis_name="s",
        num_cores=NC, num_subcores=NS),
    scratch_types=[
        pltpu.VMEM((COLS,), jnp.float32),
        pltpu.VMEM((COLS,), jnp.float32),
        pltpu.VMEM((COLS,), jnp.int32),
        pltpu.VMEM((256 * L,), jnp.int32),
        pltpu.SMEM((256,), jnp.int32),
        pltpu.SemaphoreType.DMA,
        pltpu.SemaphoreType.DMA,
        pltpu.SemaphoreType.DMA,
        pltpu.SemaphoreType.DMA,
    ],
    compiler_params=pltpu.CompilerParams(needs_layout_passes=False),
)(_topk_body)


# speedup vs baseline: 58.4285x; 1.2611x over previous
import functools

import jax
import jax.numpy as jnp
from jax import lax
from jax.experimental import pallas as pl
from jax.experimental.pallas import tpu as pltpu
from jax.experimental.pallas import tpu_sc as plsc

ROWS = 4096
COLS = 32768
KTOP = 512
NC = 2
NS = 16
L = 16
NW = NC * NS
NVEC = COLS // L
RPW = ROWS // NW


def _topk_body(x_hbm, out_hbm, row_a, row_b, cand, hist, cnt,
               sem_ia, sem_ib, sem_oa, sem_ob):
  cid = lax.axis_index("c")
  sid = lax.axis_index("s")
  wid = sid * NC + cid

  zeros16 = jnp.zeros((L,), jnp.int32)
  ones16 = jnp.ones((L,), jnp.int32)
  lanes = lax.iota(jnp.int32, L)

  @plsc.parallel_loop(0, 256, unroll=4)
  def _(b):
    hist[pl.ds(b * L, L)] = zeros16

  def threshold(row_buf):
    @plsc.parallel_loop(0, NVEC, unroll=8)
    def _(i):
      v = row_buf[pl.ds(i * L, L)]
      bu = plsc.bitcast(v, jnp.int32)
      idx = (lax.shift_right_logical(bu, 20) & jnp.int32(0xFF0)) | lanes
      plsc.addupdate_scatter(hist, [idx], ones16)

    @plsc.parallel_loop(0, 256, unroll=4)
    def _(b):
      cnt[b] = jnp.sum(hist[pl.ds(b * L, L)])
      hist[pl.ds(b * L, L)] = zeros16

    def cond1(st):
      k, acc = st
      b = jnp.where(k < 128, 127 - k, k)
      return acc + cnt[b] < KTOP

    def body1(st):
      k, acc = st
      b = jnp.where(k < 128, 127 - k, k)
      return k + 1, acc + cnt[b]

    kstar, acc_above = lax.while_loop(cond1, body1,
                                      (jnp.int32(0), jnp.int32(0)))
    b1raw = jnp.where(kstar < 128, 127 - kstar, kstar)
    rank = jnp.int32(KTOP) - acc_above
    prefix = b1raw

    @plsc.parallel_loop(0, NVEC, unroll=8, carry=jnp.int32(0))
    def ncand(i, off):
      v = row_buf[pl.ds(i * L, L)]
      bu = plsc.bitcast(v, jnp.int32)
      raw = lax.shift_right_logical(bu, 24)
      m = raw == b1raw
      plsc.store_compressed(cand.at[pl.ds(off, L)], bu, mask=m)
      return off + plsc.all_reduce_population_count(m)[0]

    nv = (ncand + (L - 1)) // L
    is_pos = b1raw < 128

    def round_body(ri, st):
      prefix, rank = st
      sh = 20 - 4 * ri

      @plsc.parallel_loop(0, nv, unroll=2)
      def _(i):
        s = cand[pl.ds(i * L, L)]
        valid = (i * L + lanes) < ncand
        m = valid & (lax.shift_right_logical(s, sh + 4) == prefix)
        idx = ((lax.shift_right_logical(s, sh) & jnp.int32(0xF)) * L) | lanes
        plsc.addupdate_scatter(hist, [idx], ones16, mask=m)

      for b in range(16):
        cnt[b] = jnp.sum(hist[pl.ds(b * L, L)])
        hist[pl.ds(b * L, L)] = zeros16

      def cond(cs_):
        w, a = cs_
        b = jnp.where(is_pos, 15 - w, w)
        return a + cnt[b] < rank

      def body(cs_):
        w, a = cs_
        b = jnp.where(is_pos, 15 - w, w)
        return w + 1, a + cnt[b]

      wstar, acc_ab = lax.while_loop(cond, body,
                                     (jnp.int32(0), jnp.int32(0)))
      bstar = jnp.where(is_pos, 15 - wstar, wstar)
      return prefix * 16 + bstar, rank - acc_ab

    prefix, rank = lax.fori_loop(0, 6, round_body, (prefix, rank))
    t_key = jnp.where(prefix < 0, prefix ^ jnp.int32(0x7FFFFFFF), prefix)
    return jnp.maximum(t_key, 1)

  def mask_row(row_buf, t):
    @plsc.parallel_loop(0, NVEC, unroll=8)
    def _(i):
      v = row_buf[pl.ds(i * L, L)]
      bu = plsc.bitcast(v, jnp.int32)
      row_buf[pl.ds(i * L, L)] = jnp.where(bu >= t, v, jnp.float32(0.0))

  pltpu.async_copy(x_hbm.at[wid], row_a, sem_ia)

  def do_pair(jj, carry):
    r0 = wid + (2 * jj) * NW
    r1 = r0 + NW
    pltpu.make_async_copy(x_hbm.at[r0], row_a, sem_ia).wait()

    @pl.when(jj > 0)
    def _():
      pltpu.make_async_copy(row_b, out_hbm.at[r1 - 2 * NW], sem_ob).wait()

    pltpu.async_copy(x_hbm.at[r1], row_b, sem_ib)

    t0 = threshold(row_a)
    mask_row(row_a, t0)
    pltpu.async_copy(row_a, out_hbm.at[r0], sem_oa)

    pltpu.make_async_copy(x_hbm.at[r1], row_b, sem_ib).wait()
    t1 = threshold(row_b)

    pltpu.make_async_copy(row_a, out_hbm.at[r0], sem_oa).wait()

    @pl.when(jj < RPW // 2 - 1)
    def _():
      pltpu.async_copy(x_hbm.at[r0 + 2 * NW], row_a, sem_ia)

    mask_row(row_b, t1)
    pltpu.async_copy(row_b, out_hbm.at[r1], sem_ob)
    return carry

  lax.fori_loop(0, RPW // 2, do_pair, 0)
  last_r1 = wid + (RPW - 1) * NW
  pltpu.make_async_copy(row_b, out_hbm.at[last_r1], sem_ob).wait()


kernel = functools.partial(
    pl.kernel,
    out_type=jax.ShapeDtypeStruct((ROWS, COLS), jnp.float32),
    mesh=plsc.VectorSubcoreMesh(
        core_axis_name="c", subcore_axis_name="s",
        num_cores=NC, num_subcores=NS),
    scratch_types=[
        pltpu.VMEM((COLS,), jnp.float32),
        pltpu.VMEM((COLS,), jnp.float32),
        pltpu.VMEM((COLS,), jnp.int32),
        pltpu.VMEM((256 * L,), jnp.int32),
        pltpu.SMEM((256,), jnp.int32),
        pltpu.SemaphoreType.DMA,
        pltpu.SemaphoreType.DMA,
        pltpu.SemaphoreType.DMA,
        pltpu.SemaphoreType.DMA,
    ],
    compiler_params=pltpu.CompilerParams(needs_layout_passes=False),
)(_topk_body)


# speedup vs baseline: 59.1653x; 1.0126x over previous
import functools

import jax
import jax.numpy as jnp
from jax import lax
from jax.experimental import pallas as pl
from jax.experimental.pallas import tpu as pltpu
from jax.experimental.pallas import tpu_sc as plsc

ROWS = 4096
COLS = 32768
KTOP = 512
NC = 2
NS = 16
L = 16
NW = NC * NS
NVEC = COLS // L
RPW = ROWS // NW


def _topk_body(x_hbm, out_hbm, row_a, row_b, cand, hist, cnt,
               sem_ia, sem_ib, sem_oa, sem_ob):
  cid = lax.axis_index("c")
  sid = lax.axis_index("s")
  wid = sid * NC + cid

  zeros16 = jnp.zeros((L,), jnp.int32)
  ones16 = jnp.ones((L,), jnp.int32)
  lanes = lax.iota(jnp.int32, L)

  @plsc.parallel_loop(0, 256, unroll=4)
  def _(b):
    hist[pl.ds(b * L, L)] = zeros16

  def threshold(row_buf):
    @plsc.parallel_loop(0, NVEC, unroll=16)
    def _(i):
      v = row_buf[pl.ds(i * L, L)]
      bu = plsc.bitcast(v, jnp.int32)
      idx = (lax.shift_right_logical(bu, 20) & jnp.int32(0xFF0)) | lanes
      plsc.addupdate_scatter(hist, [idx], ones16)

    @plsc.parallel_loop(0, 256, unroll=8)
    def _(b):
      cnt[b] = jnp.sum(hist[pl.ds(b * L, L)])
      hist[pl.ds(b * L, L)] = zeros16

    def cond1(st):
      k, acc = st
      b = jnp.where(k < 128, 127 - k, k)
      return acc + cnt[b] < KTOP

    def body1(st):
      k, acc = st
      b = jnp.where(k < 128, 127 - k, k)
      return k + 1, acc + cnt[b]

    kstar, acc_above = lax.while_loop(cond1, body1,
                                      (jnp.int32(0), jnp.int32(0)))
    b1raw = jnp.where(kstar < 128, 127 - kstar, kstar)
    rank = jnp.int32(KTOP) - acc_above
    prefix = b1raw

    @plsc.parallel_loop(0, NVEC, unroll=8, carry=jnp.int32(0))
    def ncand(i, off):
      v = row_buf[pl.ds(i * L, L)]
      bu = plsc.bitcast(v, jnp.int32)
      raw = lax.shift_right_logical(bu, 24)
      m = raw == b1raw
      plsc.store_compressed(cand.at[pl.ds(off, L)], bu, mask=m)
      return off + plsc.all_reduce_population_count(m)[0]

    nv = (ncand + (L - 1)) // L
    is_pos = b1raw < 128

    def round_body(ri, st):
      prefix, rank = st
      sh = 20 - 4 * ri

      @plsc.parallel_loop(0, nv, unroll=2)
      def _(i):
        s = cand[pl.ds(i * L, L)]
        valid = (i * L + lanes) < ncand
        m = valid & (lax.shift_right_logical(s, sh + 4) == prefix)
        idx = ((lax.shift_right_logical(s, sh) & jnp.int32(0xF)) * L) | lanes
        plsc.addupdate_scatter(hist, [idx], ones16, mask=m)

      for b in range(16):
        cnt[b] = jnp.sum(hist[pl.ds(b * L, L)])
        hist[pl.ds(b * L, L)] = zeros16

      def cond(cs_):
        w, a = cs_
        b = jnp.where(is_pos, 15 - w, w)
        return a + cnt[b] < rank

      def body(cs_):
        w, a = cs_
        b = jnp.where(is_pos, 15 - w, w)
        return w + 1, a + cnt[b]

      wstar, acc_ab = lax.while_loop(cond, body,
                                     (jnp.int32(0), jnp.int32(0)))
      bstar = jnp.where(is_pos, 15 - wstar, wstar)
      return prefix * 16 + bstar, rank - acc_ab

    prefix, rank = lax.fori_loop(0, 6, round_body, (prefix, rank))
    t_key = jnp.where(prefix < 0, prefix ^ jnp.int32(0x7FFFFFFF), prefix)
    return jnp.maximum(t_key, 1)

  def mask_row(row_buf, t):
    @plsc.parallel_loop(0, NVEC, unroll=16)
    def _(i):
      v = row_buf[pl.ds(i * L, L)]
      bu = plsc.bitcast(v, jnp.int32)
      row_buf[pl.ds(i * L, L)] = jnp.where(bu >= t, v, jnp.float32(0.0))

  pltpu.async_copy(x_hbm.at[wid], row_a, sem_ia)

  def do_pair(jj, carry):
    r0 = wid + (2 * jj) * NW
    r1 = r0 + NW
    pltpu.make_async_copy(x_hbm.at[r0], row_a, sem_ia).wait()

    @pl.when(jj > 0)
    def _():
      pltpu.make_async_copy(row_b, out_hbm.at[r1 - 2 * NW], sem_ob).wait()

    pltpu.async_copy(x_hbm.at[r1], row_b, sem_ib)

    t0 = threshold(row_a)
    mask_row(row_a, t0)
    pltpu.async_copy(row_a, out_hbm.at[r0], sem_oa)

    pltpu.make_async_copy(x_hbm.at[r1], row_b, sem_ib).wait()
    t1 = threshold(row_b)

    pltpu.make_async_copy(row_a, out_hbm.at[r0], sem_oa).wait()

    @pl.when(jj < RPW // 2 - 1)
    def _():
      pltpu.async_copy(x_hbm.at[r0 + 2 * NW], row_a, sem_ia)

    mask_row(row_b, t1)
    pltpu.async_copy(row_b, out_hbm.at[r1], sem_ob)
    return carry

  lax.fori_loop(0, RPW // 2, do_pair, 0)
  last_r1 = wid + (RPW - 1) * NW
  pltpu.make_async_copy(row_b, out_hbm.at[last_r1], sem_ob).wait()


kernel = functools.partial(
    pl.kernel,
    out_type=jax.ShapeDtypeStruct((ROWS, COLS), jnp.float32),
    mesh=plsc.VectorSubcoreMesh(
        core_axis_name="c", subcore_axis_name="s",
        num_cores=NC, num_subcores=NS),
    scratch_types=[
        pltpu.VMEM((COLS,), jnp.float32),
        pltpu.VMEM((COLS,), jnp.float32),
        pltpu.VMEM((COLS,), jnp.int32),
        pltpu.VMEM((256 * L,), jnp.int32),
        pltpu.SMEM((256,), jnp.int32),
        pltpu.SemaphoreType.DMA,
        pltpu.SemaphoreType.DMA,
        pltpu.SemaphoreType.DMA,
        pltpu.SemaphoreType.DMA,
    ],
    compiler_params=pltpu.CompilerParams(needs_layout_passes=False),
)(_topk_body)
